# Initial kernel scaffold; baseline (speedup 1.0000x reference)
import jax
import jax.numpy as jnp
from jax.experimental import pallas as pl


def kernel(spec, window):
    raise NotImplementedError("write your pallas kernel here")


# speedup vs baseline: 48.7855x; 48.7855x over previous
import functools

import jax
import jax.numpy as jnp
import numpy as np
from jax.experimental import pallas as pl

N_FFT = 1024
HOP = 256
WIN = 1024
EPS = 1e-11
NFREQ = N_FFT // 2 + 1
TAPS = WIN // HOP
BM = 128


def _idft_matrix() -> np.ndarray:
    k = np.arange(NFREQ, dtype=np.float64)
    n = np.arange(N_FFT, dtype=np.float64)
    coef = np.full(NFREQ, 2.0)
    coef[0] = 1.0
    coef[NFREQ - 1] = 1.0
    m = (coef[None, :] * np.cos(2.0 * np.pi * np.outer(n, k) / N_FFT)) / N_FFT
    return m.astype(np.float32)


def _istft_kernel(t_total, p0_ref, p1_ref, m_ref, w_ref, out_ref):
    k = pl.program_id(0)
    a = m_ref[...] * w_ref[...]
    x = jnp.concatenate([p0_ref[...], p1_ref[...]], axis=1)
    b = x.shape[0]
    acc = jnp.zeros((b * BM, HOP), dtype=jnp.float32)
    for j in range(TAPS):
        xs = x[:, 3 - j:3 - j + BM, :].reshape(b * BM, NFREQ)
        aj = a[j * HOP:(j + 1) * HOP, :]
        acc = acc + jax.lax.dot_general(
            xs, aj, (((1,), (1,)), ((), ())),
            preferred_element_type=jnp.float32)
    m_idx = k * BM + jax.lax.broadcasted_iota(jnp.int32, (BM, 1), 0)
    wsq = (w_ref[...] * w_ref[...]).reshape(TAPS, HOP)
    env = jnp.zeros((BM, HOP), dtype=jnp.float32)
    for j in range(TAPS):
        t = m_idx - j
        valid = jnp.logical_and(t >= 0, t < t_total).astype(jnp.float32)
        env = env + valid * wsq[j][None, :]
    y = acc.reshape(b, BM, HOP) / (env + EPS)[None]
    out_ref[...] = y


@jax.jit
def kernel(spec, window):
    b, nfreq, t = spec.shape
    n_blocks = t + TAPS - 1
    n_chunks = pl.cdiv(n_blocks, BM)
    mpad = n_chunks * BM
    spec_t = jnp.swapaxes(spec, 1, 2)
    p = jnp.zeros((b, mpad + BM, nfreq), dtype=spec.dtype)
    p = jax.lax.dynamic_update_slice(p, spec_t, (0, TAPS - 1, 0))

    m = jnp.asarray(_idft_matrix())
    w2d = window.reshape(WIN, 1)

    out = pl.pallas_call(
        functools.partial(_istft_kernel, t),
        grid=(n_chunks,),
        in_specs=[
            pl.BlockSpec((b, BM, nfreq), lambda k: (0, k, 0)),
            pl.BlockSpec((b, BM, nfreq), lambda k: (0, k + 1, 0)),
            pl.BlockSpec((WIN, NFREQ), lambda k: (0, 0)),
            pl.BlockSpec((WIN, 1), lambda k: (0, 0)),
        ],
        out_specs=pl.BlockSpec((b, BM, HOP), lambda k: (0, k, 0)),
        out_shape=jax.ShapeDtypeStruct((b, mpad, HOP), jnp.float32),
    )(p, p, m, w2d)

    pad = (WIN - HOP) // 2
    y = out.reshape(b, mpad * HOP)
    return jax.lax.dynamic_slice(y, (0, pad), (b, (t - 1) * HOP + WIN - 2 * pad))


# speedup vs baseline: 56.8585x; 1.1655x over previous
import functools

import jax
import jax.numpy as jnp
import numpy as np
from jax.experimental import pallas as pl

N_FFT = 1024
HOP = 256
WIN = 1024
EPS = 1e-11
NFREQ = N_FFT // 2 + 1
TAPS = WIN // HOP
BM = 128


def _idft_matrix() -> np.ndarray:
    k = np.arange(NFREQ, dtype=np.float64)
    n = np.arange(N_FFT, dtype=np.float64)
    coef = np.full(NFREQ, 2.0)
    coef[0] = 1.0
    coef[NFREQ - 1] = 1.0
    m = (coef[None, :] * np.cos(2.0 * np.pi * np.outer(n, k) / N_FFT)) / N_FFT
    return m.astype(np.float32)


def _istft_kernel(t_total, p0_ref, p1_ref, m_ref, w_ref, out_ref):
    k = pl.program_id(0)
    a = m_ref[...] * w_ref[...]
    x = jnp.concatenate([p0_ref[...], p1_ref[...]], axis=2)
    b = x.shape[0]
    m_idx = k * BM + jax.lax.broadcasted_iota(jnp.int32, (1, BM), 1)
    wsq = (w_ref[...] * w_ref[...]).reshape(TAPS, HOP)
    env = jnp.zeros((HOP, BM), dtype=jnp.float32)
    masks = []
    for j in range(TAPS):
        t = m_idx - j
        valid = jnp.logical_and(t >= 0, t < t_total).astype(jnp.float32)
        masks.append(valid)
        env = env + valid * wsq[j][:, None]
    inv_env = 1.0 / (env + EPS)
    for bi in range(b):
        acc = jnp.zeros((HOP, BM), dtype=jnp.float32)
        for j in range(TAPS):
            xs = x[bi, :, BM - j:2 * BM - j]
            aj = a[j * HOP:(j + 1) * HOP, :]
            acc = acc + masks[j] * jax.lax.dot_general(
                aj, xs, (((1,), (0,)), ((), ())),
                preferred_element_type=jnp.float32)
        out_ref[bi, :, :] = acc * inv_env


@jax.jit
def kernel(spec, window):
    b, nfreq, t = spec.shape
    n_blocks = t + TAPS - 1
    n_chunks = pl.cdiv(n_blocks, BM)
    mpad = n_chunks * BM
    t_blocks = t // BM

    m = jnp.asarray(_idft_matrix())
    w2d = window.reshape(WIN, 1)

    def idx_lo(k):
        return (0, 0, jnp.clip(k - 1, 0, t_blocks - 1))

    def idx_hi(k):
        return (0, 0, jnp.clip(k, 0, t_blocks - 1))

    out = pl.pallas_call(
        functools.partial(_istft_kernel, t),
        grid=(n_chunks,),
        in_specs=[
            pl.BlockSpec((b, nfreq, BM), idx_lo),
            pl.BlockSpec((b, nfreq, BM), idx_hi),
            pl.BlockSpec((WIN, NFREQ), lambda k: (0, 0)),
            pl.BlockSpec((WIN, 1), lambda k: (0, 0)),
        ],
        out_specs=pl.BlockSpec((b, HOP, BM), lambda k: (0, 0, k)),
        out_shape=jax.ShapeDtypeStruct((b, HOP, mpad), jnp.float32),
    )(spec, spec, m, w2d)

    pad = (WIN - HOP) // 2
    y = jnp.swapaxes(out, 1, 2).reshape(b, mpad * HOP)
    return jax.lax.dynamic_slice(y, (0, pad), (b, (t - 1) * HOP + WIN - 2 * pad))


# speedup vs baseline: 65.8493x; 1.1581x over previous
import functools

import jax
import jax.numpy as jnp
import numpy as np
from jax.experimental import pallas as pl

N_FFT = 1024
HOP = 256
WIN = 1024
EPS = 1e-11
NFREQ = N_FFT // 2 + 1
TAPS = WIN // HOP
BM = 256


def _idft_matrix() -> np.ndarray:
    k = np.arange(NFREQ, dtype=np.float64)
    n = np.arange(N_FFT, dtype=np.float64)
    coef = np.full(NFREQ, 2.0)
    coef[0] = 1.0
    coef[NFREQ - 1] = 1.0
    m = (coef[None, :] * np.cos(2.0 * np.pi * np.outer(n, k) / N_FFT)) / N_FFT
    return m.astype(np.float32)


def _istft_kernel(t_total, p0_ref, p1_ref, m_ref, w_ref, out_ref):
    k = pl.program_id(0)
    a = m_ref[...] * w_ref[...]
    x = jnp.concatenate([p0_ref[...], p1_ref[...]], axis=2)
    b = x.shape[0]
    m_idx = k * BM + jax.lax.broadcasted_iota(jnp.int32, (1, BM), 1)
    wsq = (w_ref[...] * w_ref[...]).reshape(TAPS, HOP)
    env = jnp.zeros((HOP, BM), dtype=jnp.float32)
    masks = []
    for j in range(TAPS):
        t = m_idx - j
        valid = jnp.logical_and(t >= 0, t < t_total).astype(jnp.float32)
        masks.append(valid)
        env = env + valid * wsq[j][:, None]
    inv_env = 1.0 / (env + EPS)
    for bi in range(b):
        acc = jnp.zeros((HOP, BM), dtype=jnp.float32)
        for j in range(TAPS):
            xs = x[bi, :, BM - j:2 * BM - j]
            aj = a[j * HOP:(j + 1) * HOP, :]
            acc = acc + masks[j] * jax.lax.dot_general(
                aj, xs, (((1,), (0,)), ((), ())),
                preferred_element_type=jnp.float32)
        out_ref[bi, :, :] = acc * inv_env


@jax.jit
def kernel(spec, window):
    b, nfreq, t = spec.shape
    n_blocks = t + TAPS - 1
    n_chunks = pl.cdiv(n_blocks, BM)
    mpad = n_chunks * BM
    t_blocks = t // BM

    m = jnp.asarray(_idft_matrix())
    w2d = window.reshape(WIN, 1)

    def idx_lo(k):
        return (0, 0, jnp.clip(k - 1, 0, t_blocks - 1))

    def idx_hi(k):
        return (0, 0, jnp.clip(k, 0, t_blocks - 1))

    out = pl.pallas_call(
        functools.partial(_istft_kernel, t),
        grid=(n_chunks,),
        in_specs=[
            pl.BlockSpec((b, nfreq, BM), idx_lo),
            pl.BlockSpec((b, nfreq, BM), idx_hi),
            pl.BlockSpec((WIN, NFREQ), lambda k: (0, 0)),
            pl.BlockSpec((WIN, 1), lambda k: (0, 0)),
        ],
        out_specs=pl.BlockSpec((b, HOP, BM), lambda k: (0, 0, k)),
        out_shape=jax.ShapeDtypeStruct((b, HOP, mpad), jnp.float32),
    )(spec, spec, m, w2d)

    pad = (WIN - HOP) // 2
    y = jnp.swapaxes(out, 1, 2).reshape(b, mpad * HOP)
    return jax.lax.dynamic_slice(y, (0, pad), (b, (t - 1) * HOP + WIN - 2 * pad))
